# baseline (speedup 1.0000x reference)
import jax
import jax.numpy as jnp
from jax.experimental import pallas as pl
from jax.experimental.pallas import tpu as pltpu
from jax.experimental.pallas import tpu_sc as plsc

T = 1024
D = 768
I = 3072
E = 8
CAP = T
TM = 128
NKW = 1536
NCH = I // NKW
SCW = 128

_f32 = jnp.float32
_bf16 = jnp.bfloat16


def _router_body(y_ref, dest_ref, w0_ref, w1_ref, cnt_ref):
    y = y_ref[...]
    iota = jax.lax.broadcasted_iota(jnp.int32, (T, E), 1)
    v0 = jnp.max(y, axis=1, keepdims=True)
    i0 = jnp.min(jnp.where(y == v0, iota, E), axis=1, keepdims=True)
    y2 = jnp.where(iota == i0, -jnp.inf, y)
    v1 = jnp.max(y2, axis=1, keepdims=True)
    i1 = jnp.min(jnp.where(y2 == v1, iota, E), axis=1, keepdims=True)
    e1 = jnp.exp(v1 - v0)
    den = 1.0 + e1
    a0 = (iota == i0).astype(_f32)
    a1 = (iota == i1).astype(_f32)
    rows = jax.lax.broadcasted_iota(jnp.int32, (T, T), 0)
    cols = jax.lax.broadcasted_iota(jnp.int32, (T, T), 1)
    tri = (cols < rows).astype(_f32)
    c0 = jax.lax.dot_general(tri, a0, (((1,), (0,)), ((), ())),
                             preferred_element_type=_f32)
    c1 = jax.lax.dot_general(tri, a1, (((1,), (0,)), ((), ())),
                             preferred_element_type=_f32)
    s0 = jnp.sum(a0, axis=0, keepdims=True)
    s1 = jnp.sum(a1, axis=0, keepdims=True)
    r0 = jnp.sum(c0 * a0, axis=1)
    r1 = jnp.sum((c1 + s0) * a1, axis=1)
    dest_ref[0, :] = i0[:, 0] * CAP + r0.astype(jnp.int32)
    dest_ref[1, :] = i1[:, 0] * CAP + r1.astype(jnp.int32)
    w0_ref[...] = 1.0 / den
    w1_ref[...] = e1 / den
    cnt_ref[0, :] = (s0 + s1)[0, :].astype(jnp.int32)


def _ffn_body(cnt_ref, xs_ref, w1a_ref, w1b_ref, wo_ref, bina_ref, binb_ref,
              bout_ref, ff_ref, w1a16_ref, w1b16_ref, wo16_ref):
    n = pl.program_id(1)
    w1a16_ref[...] = w1a_ref[...].astype(_bf16)
    w1b16_ref[...] = w1b_ref[...].astype(_bf16)
    wo16_ref[...] = wo_ref[...].astype(_bf16)
    c = cnt_ref[pl.program_id(0)]
    nt = (c + TM - 1) // TM

    def tile(mi, carry):
        sl = pl.ds(mi * TM, TM)
        xv = xs_ref[sl, :]
        h1 = jax.lax.dot_general(xv, w1a16_ref[...], (((1,), (1,)), ((), ())),
                                 preferred_element_type=_f32) + bina_ref[...]
        h2 = jax.lax.dot_general(xv, w1b16_ref[...], (((1,), (1,)), ((), ())),
                                 preferred_element_type=_f32) + binb_ref[...]
        act = (h1 * (1.0 / (1.0 + jnp.exp(-h2)))).astype(_bf16)
        o = jax.lax.dot_general(act, wo16_ref[...], (((1,), (1,)), ((), ())),
                                preferred_element_type=_f32)

        @pl.when(n == 0)
        def _():
            ff_ref[sl, :] = (o + bout_ref[...]).astype(_bf16)

        @pl.when(n > 0)
        def _():
            ff_ref[sl, :] = (ff_ref[sl, :].astype(_f32) + o).astype(_bf16)

        return carry

    jax.lax.fori_loop(0, nt, tile, 0)


def _comb_body(w0_ref, w1_ref, g0_ref, g1_ref, o_ref):
    o_ref[...] = (w0_ref[...] * g0_ref[...].astype(_f32)
                  + w1_ref[...] * g1_ref[...].astype(_f32))


def kernel(x, W_r, b_r, W_in, b_in, W_out, b_out):
    logits = x @ W_r.T + b_r
    u = jax.random.uniform(jax.random.key(42), logits.shape, dtype=logits.dtype)
    noise = -jnp.log(-jnp.log(u + 1e-20) + 1e-20)
    y = (logits + noise) / 1.0

    dest, w0c, w1c, cnt = pl.pallas_call(
        _router_body,
        out_shape=(
            jax.ShapeDtypeStruct((2, T), jnp.int32),
            jax.ShapeDtypeStruct((T, 1), _f32),
            jax.ShapeDtypeStruct((T, 1), _f32),
            jax.ShapeDtypeStruct((1, E), jnp.int32),
        ),
    )(y)
    counts = cnt.reshape(E)
    idx = dest.reshape(2 * T // SCW, SCW)

    vmesh = plsc.VectorSubcoreMesh(core_axis_name="c", subcore_axis_name="s",
                                   num_cores=2, num_subcores=16)
    x32 = jax.lax.bitcast_convert_type(
        x.astype(_bf16).reshape(T, D // 2, 2), jnp.int32)

    @pl.kernel(out_type=jax.ShapeDtypeStruct((E * CAP, D // 2), jnp.int32),
               mesh=vmesh)
    def _scatter_k(x_hbm, i_hbm, xs_hbm):
        def body(x_vmem, i_vmem):
            pltpu.sync_copy(x_vmem, xs_hbm.at[i_vmem.at[0]])

        pltpu.emit_pipeline(
            body,
            grid=(2 * T // SCW,),
            in_specs=[
                pl.BlockSpec((SCW, D // 2), lambda i: (i % (T // SCW), 0)),
                pl.BlockSpec((1, SCW), lambda i: (i, 0)),
            ],
            out_specs=[],
            core_axis_name=("c", "s"),
            dimension_semantics=(pltpu.PARALLEL,),
        )(x_hbm, i_hbm)

    xs = jax.lax.bitcast_convert_type(
        _scatter_k(x32, idx), _bf16).reshape(E * CAP, D)

    grid_spec = pltpu.PrefetchScalarGridSpec(
        num_scalar_prefetch=1,
        grid=(E, NCH),
        in_specs=[
            pl.BlockSpec((CAP, D), lambda e, n, c: (e, 0)),
            pl.BlockSpec((None, NKW, D), lambda e, n, c: (e, n, 0)),
            pl.BlockSpec((None, NKW, D), lambda e, n, c: (e, NCH + n, 0)),
            pl.BlockSpec((None, D, NKW), lambda e, n, c: (e, 0, n)),
            pl.BlockSpec((None, 1, NKW), lambda e, n, c: (e * 2 * NCH + n, 0, 0)),
            pl.BlockSpec((None, 1, NKW),
                         lambda e, n, c: (e * 2 * NCH + NCH + n, 0, 0)),
            pl.BlockSpec((None, 1, D), lambda e, n, c: (e, 0, 0)),
        ],
        out_specs=pl.BlockSpec((CAP, D), lambda e, n, c: (e, 0)),
        scratch_shapes=[pltpu.VMEM((NKW, D), _bf16),
                        pltpu.VMEM((NKW, D), _bf16),
                        pltpu.VMEM((D, NKW), _bf16)],
    )
    ff = pl.pallas_call(
        _ffn_body,
        grid_spec=grid_spec,
        out_shape=jax.ShapeDtypeStruct((E * CAP, D), _bf16),
    )(counts, xs, W_in, W_in, W_out,
      b_in.reshape(E * 2 * NCH, 1, NKW), b_in.reshape(E * 2 * NCH, 1, NKW),
      b_out.reshape(E, 1, D))

    ff32 = jax.lax.bitcast_convert_type(ff.reshape(E * CAP, D // 2, 2),
                                        jnp.int32)

    @pl.kernel(out_type=jax.ShapeDtypeStruct((2 * T, D // 2), jnp.int32),
               mesh=vmesh)
    def _gather_k(ff_hbm, i_hbm, g_hbm):
        def body(i_vmem, g_vmem):
            pltpu.sync_copy(ff_hbm.at[i_vmem.at[0]], g_vmem)

        pltpu.emit_pipeline(
            body,
            grid=(2 * T // SCW,),
            in_specs=[pl.BlockSpec((1, SCW), lambda i: (i, 0))],
            out_specs=[pl.BlockSpec((SCW, D // 2), lambda i: (i, 0))],
            core_axis_name=("c", "s"),
            dimension_semantics=(pltpu.PARALLEL,),
        )(i_hbm, g_hbm)

    g = jax.lax.bitcast_convert_type(_gather_k(ff32, idx),
                                     _bf16).reshape(2 * T, D)

    out = pl.pallas_call(
        _comb_body,
        out_shape=jax.ShapeDtypeStruct((T, D), _f32),
    )(w0c, w1c, g[:T], g[T:])
    return out


# speedup vs baseline: 1.8063x; 1.8063x over previous
import jax
import jax.numpy as jnp
from jax.experimental import pallas as pl
from jax.experimental.pallas import tpu as pltpu
from jax.experimental.pallas import tpu_sc as plsc

T = 1024
D = 768
I = 3072
E = 8
CAP = T
TM = 128
NKW = 1536
NCH = I // NKW
SCW = 128

_f32 = jnp.float32
_bf16 = jnp.bfloat16


def _router_body(y_ref, dest_ref, w0_ref, w1_ref, cnt_ref):
    y = y_ref[...]
    iota = jax.lax.broadcasted_iota(jnp.int32, (T, E), 1)
    v0 = jnp.max(y, axis=1, keepdims=True)
    i0 = jnp.min(jnp.where(y == v0, iota, E), axis=1, keepdims=True)
    y2 = jnp.where(iota == i0, -jnp.inf, y)
    v1 = jnp.max(y2, axis=1, keepdims=True)
    i1 = jnp.min(jnp.where(y2 == v1, iota, E), axis=1, keepdims=True)
    e1 = jnp.exp(v1 - v0)
    den = 1.0 + e1
    a0 = (iota == i0).astype(_f32)
    a1 = (iota == i1).astype(_f32)
    rows = jax.lax.broadcasted_iota(jnp.int32, (T, T), 0)
    cols = jax.lax.broadcasted_iota(jnp.int32, (T, T), 1)
    tri = (cols < rows).astype(_f32)
    c0 = jax.lax.dot_general(tri, a0, (((1,), (0,)), ((), ())),
                             preferred_element_type=_f32)
    c1 = jax.lax.dot_general(tri, a1, (((1,), (0,)), ((), ())),
                             preferred_element_type=_f32)
    s0 = jnp.sum(a0, axis=0, keepdims=True)
    s1 = jnp.sum(a1, axis=0, keepdims=True)
    r0 = jnp.sum(c0 * a0, axis=1)
    r1 = jnp.sum((c1 + s0) * a1, axis=1)
    dest_ref[0, :] = i0[:, 0] * CAP + r0.astype(jnp.int32)
    dest_ref[1, :] = i1[:, 0] * CAP + r1.astype(jnp.int32)
    w0_ref[...] = 1.0 / den
    w1_ref[...] = e1 / den
    cnt_ref[0, :] = (s0 + s1)[0, :].astype(jnp.int32)


def _ffn_body(cnt_ref, xs_ref, w1a_ref, w1b_ref, wo_ref, bina_ref, binb_ref,
              bout_ref, ff_ref, w1a16_ref, w1b16_ref, wo16_ref):
    n = pl.program_id(1)
    w1a16_ref[...] = w1a_ref[...].astype(_bf16)
    w1b16_ref[...] = w1b_ref[...].astype(_bf16)
    wo16_ref[...] = wo_ref[...].astype(_bf16)
    c = cnt_ref[pl.program_id(0)]
    nt = (c + TM - 1) // TM

    def tile(mi, carry):
        sl = pl.ds(mi * TM, TM)
        xv = xs_ref[sl, :].astype(_bf16)
        h1 = jax.lax.dot_general(xv, w1a16_ref[...], (((1,), (1,)), ((), ())),
                                 preferred_element_type=_f32) + bina_ref[...]
        h2 = jax.lax.dot_general(xv, w1b16_ref[...], (((1,), (1,)), ((), ())),
                                 preferred_element_type=_f32) + binb_ref[...]
        act = (h1 * (1.0 / (1.0 + jnp.exp(-h2)))).astype(_bf16)
        o = jax.lax.dot_general(act, wo16_ref[...], (((1,), (1,)), ((), ())),
                                preferred_element_type=_f32)

        @pl.when(n == 0)
        def _():
            ff_ref[sl, :] = o + bout_ref[...]

        @pl.when(n > 0)
        def _():
            ff_ref[sl, :] += o

        return carry

    jax.lax.fori_loop(0, nt, tile, 0)


def _comb_body(w0_ref, w1_ref, g0_ref, g1_ref, o_ref):
    o_ref[...] = w0_ref[...] * g0_ref[...] + w1_ref[...] * g1_ref[...]


def kernel(x, W_r, b_r, W_in, b_in, W_out, b_out):
    logits = x @ W_r.T + b_r
    u = jax.random.uniform(jax.random.key(42), logits.shape, dtype=logits.dtype)
    noise = -jnp.log(-jnp.log(u + 1e-20) + 1e-20)
    y = (logits + noise) / 1.0

    dest, w0c, w1c, cnt = pl.pallas_call(
        _router_body,
        out_shape=(
            jax.ShapeDtypeStruct((2, T), jnp.int32),
            jax.ShapeDtypeStruct((T, 1), _f32),
            jax.ShapeDtypeStruct((T, 1), _f32),
            jax.ShapeDtypeStruct((1, E), jnp.int32),
        ),
    )(y)
    counts = cnt.reshape(E)
    idx2 = (2 * dest.reshape(2 * T, 1)
            + jax.lax.broadcasted_iota(jnp.int32, (2 * T, 2), 1))
    idx2 = idx2.reshape(4 * T // SCW, SCW)

    vmesh = plsc.VectorSubcoreMesh(core_axis_name="c", subcore_axis_name="s",
                                   num_cores=2, num_subcores=16)
    HD = D // 2

    @pl.kernel(out_type=jax.ShapeDtypeStruct((2 * E * CAP, HD), _f32),
               mesh=vmesh)
    def _scatter_k(x_hbm, i_hbm, xs_hbm):
        def body(x_vmem, i_vmem):
            pltpu.sync_copy(x_vmem, xs_hbm.at[i_vmem.at[0]])

        pltpu.emit_pipeline(
            body,
            grid=(4 * T // SCW,),
            in_specs=[
                pl.BlockSpec((SCW, HD), lambda i: (i % (2 * T // SCW), 0)),
                pl.BlockSpec((1, SCW), lambda i: (i, 0)),
            ],
            out_specs=[],
            core_axis_name=("c", "s"),
            dimension_semantics=(pltpu.PARALLEL,),
        )(x_hbm, i_hbm)

    xs = _scatter_k(x.reshape(2 * T, HD), idx2).reshape(E * CAP, D)

    grid_spec = pltpu.PrefetchScalarGridSpec(
        num_scalar_prefetch=1,
        grid=(E, NCH),
        in_specs=[
            pl.BlockSpec((CAP, D), lambda e, n, c: (e, 0)),
            pl.BlockSpec((None, NKW, D), lambda e, n, c: (e, n, 0)),
            pl.BlockSpec((None, NKW, D), lambda e, n, c: (e, NCH + n, 0)),
            pl.BlockSpec((None, D, NKW), lambda e, n, c: (e, 0, n)),
            pl.BlockSpec((None, 1, NKW), lambda e, n, c: (e * 2 * NCH + n, 0, 0)),
            pl.BlockSpec((None, 1, NKW),
                         lambda e, n, c: (e * 2 * NCH + NCH + n, 0, 0)),
            pl.BlockSpec((None, 1, D), lambda e, n, c: (e, 0, 0)),
        ],
        out_specs=pl.BlockSpec((CAP, D), lambda e, n, c: (e, 0)),
        scratch_shapes=[pltpu.VMEM((NKW, D), _bf16),
                        pltpu.VMEM((NKW, D), _bf16),
                        pltpu.VMEM((D, NKW), _bf16)],
    )
    ff = pl.pallas_call(
        _ffn_body,
        grid_spec=grid_spec,
        out_shape=jax.ShapeDtypeStruct((E * CAP, D), _f32),
    )(counts, xs, W_in, W_in, W_out,
      b_in.reshape(E * 2 * NCH, 1, NKW), b_in.reshape(E * 2 * NCH, 1, NKW),
      b_out.reshape(E, 1, D))

    @pl.kernel(out_type=jax.ShapeDtypeStruct((4 * T, HD), _f32), mesh=vmesh)
    def _gather_k(ff_hbm, i_hbm, g_hbm):
        def body(i_vmem, g_vmem):
            pltpu.sync_copy(ff_hbm.at[i_vmem.at[0]], g_vmem)

        pltpu.emit_pipeline(
            body,
            grid=(4 * T // SCW,),
            in_specs=[pl.BlockSpec((1, SCW), lambda i: (i, 0))],
            out_specs=[pl.BlockSpec((SCW, HD), lambda i: (i, 0))],
            core_axis_name=("c", "s"),
            dimension_semantics=(pltpu.PARALLEL,),
        )(i_hbm, g_hbm)

    g = _gather_k(ff.reshape(2 * E * CAP, HD), idx2).reshape(2 * T, D)

    out = pl.pallas_call(
        _comb_body,
        out_shape=jax.ShapeDtypeStruct((T, D), _f32),
    )(w0c, w1c, g[:T], g[T:])
    return out


# speedup vs baseline: 2.0727x; 1.1475x over previous
import jax
import jax.numpy as jnp
from jax.experimental import pallas as pl
from jax.experimental.pallas import tpu as pltpu
from jax.experimental.pallas import tpu_sc as plsc

T = 1024
D = 768
I = 3072
E = 8
CAP = T
TM = 256
NKW = 1536
NCH = I // NKW
SCW = 128

_f32 = jnp.float32
_bf16 = jnp.bfloat16


def _router_body(y_ref, dest_ref, w0_ref, w1_ref, cnt_ref):
    y = y_ref[...]
    iota = jax.lax.broadcasted_iota(jnp.int32, (T, E), 1)
    v0 = jnp.max(y, axis=1, keepdims=True)
    i0 = jnp.min(jnp.where(y == v0, iota, E), axis=1, keepdims=True)
    y2 = jnp.where(iota == i0, -jnp.inf, y)
    v1 = jnp.max(y2, axis=1, keepdims=True)
    i1 = jnp.min(jnp.where(y2 == v1, iota, E), axis=1, keepdims=True)
    e1 = jnp.exp(v1 - v0)
    den = 1.0 + e1
    a0 = (iota == i0).astype(_f32)
    a1 = (iota == i1).astype(_f32)
    rows = jax.lax.broadcasted_iota(jnp.int32, (T, T), 0)
    cols = jax.lax.broadcasted_iota(jnp.int32, (T, T), 1)
    tri = (cols < rows).astype(_f32)
    c0 = jax.lax.dot_general(tri, a0, (((1,), (0,)), ((), ())),
                             preferred_element_type=_f32)
    c1 = jax.lax.dot_general(tri, a1, (((1,), (0,)), ((), ())),
                             preferred_element_type=_f32)
    s0 = jnp.sum(a0, axis=0, keepdims=True)
    s1 = jnp.sum(a1, axis=0, keepdims=True)
    r0 = jnp.sum(c0 * a0, axis=1)
    r1 = jnp.sum((c1 + s0) * a1, axis=1)
    dest_ref[0, :] = i0[:, 0] * CAP + r0.astype(jnp.int32)
    dest_ref[1, :] = i1[:, 0] * CAP + r1.astype(jnp.int32)
    w0_ref[...] = 1.0 / den
    w1_ref[...] = e1 / den
    cnt_ref[0, :] = (s0 + s1)[0, :].astype(jnp.int32)


def _ffn_body(cnt_ref, xs_ref, w1a_ref, w1b_ref, wo_ref, bina_ref, binb_ref,
              bout_ref, ff_ref, w1a16_ref, w1b16_ref, wo16_ref):
    n = pl.program_id(1)
    w1a16_ref[...] = w1a_ref[...].astype(_bf16)
    w1b16_ref[...] = w1b_ref[...].astype(_bf16)
    wo16_ref[...] = wo_ref[...].astype(_bf16)
    c = cnt_ref[pl.program_id(0)]
    nt = (c + TM - 1) // TM

    def tile(mi, carry):
        sl = pl.ds(mi * TM, TM)
        xv = xs_ref[sl, :].astype(_bf16)
        h1 = jax.lax.dot_general(xv, w1a16_ref[...], (((1,), (1,)), ((), ())),
                                 preferred_element_type=_f32) + bina_ref[...]
        h2 = jax.lax.dot_general(xv, w1b16_ref[...], (((1,), (1,)), ((), ())),
                                 preferred_element_type=_f32) + binb_ref[...]
        act = (h1 * (1.0 / (1.0 + jnp.exp(-h2)))).astype(_bf16)
        o = jax.lax.dot_general(act, wo16_ref[...], (((1,), (1,)), ((), ())),
                                preferred_element_type=_f32)

        @pl.when(n == 0)
        def _():
            ff_ref[sl, :] = o + bout_ref[...]

        @pl.when(n > 0)
        def _():
            ff_ref[sl, :] += o

        return carry

    jax.lax.fori_loop(0, nt, tile, 0)


def _comb_body(w0_ref, w1_ref, g0_ref, g1_ref, o_ref):
    o_ref[...] = w0_ref[...] * g0_ref[...] + w1_ref[...] * g1_ref[...]


def kernel(x, W_r, b_r, W_in, b_in, W_out, b_out):
    logits = x @ W_r.T + b_r
    u = jax.random.uniform(jax.random.key(42), logits.shape, dtype=logits.dtype)
    noise = -jnp.log(-jnp.log(u + 1e-20) + 1e-20)
    y = (logits + noise) / 1.0

    dest, w0c, w1c, cnt = pl.pallas_call(
        _router_body,
        out_shape=(
            jax.ShapeDtypeStruct((2, T), jnp.int32),
            jax.ShapeDtypeStruct((T, 1), _f32),
            jax.ShapeDtypeStruct((T, 1), _f32),
            jax.ShapeDtypeStruct((1, E), jnp.int32),
        ),
    )(y)
    counts = cnt.reshape(E)
    idx2 = (2 * dest.reshape(2 * T, 1)
            + jax.lax.broadcasted_iota(jnp.int32, (2 * T, 2), 1))
    idx2 = idx2.reshape(4 * T // SCW, SCW)

    vmesh = plsc.VectorSubcoreMesh(core_axis_name="c", subcore_axis_name="s",
                                   num_cores=2, num_subcores=16)
    HD = D // 2

    @pl.kernel(out_type=jax.ShapeDtypeStruct((2 * E * CAP, HD), _f32),
               mesh=vmesh)
    def _scatter_k(x_hbm, i_hbm, xs_hbm):
        def body(x_vmem, i_vmem):
            pltpu.sync_copy(x_vmem, xs_hbm.at[i_vmem.at[0]])

        pltpu.emit_pipeline(
            body,
            grid=(4 * T // SCW,),
            in_specs=[
                pl.BlockSpec((SCW, HD), lambda i: (i % (2 * T // SCW), 0)),
                pl.BlockSpec((1, SCW), lambda i: (i, 0)),
            ],
            out_specs=[],
            core_axis_name=("c", "s"),
            dimension_semantics=(pltpu.PARALLEL,),
        )(x_hbm, i_hbm)

    xs = _scatter_k(x.reshape(2 * T, HD), idx2).reshape(E * CAP, D)

    grid_spec = pltpu.PrefetchScalarGridSpec(
        num_scalar_prefetch=1,
        grid=(E, NCH),
        in_specs=[
            pl.BlockSpec((CAP, D), lambda e, n, c: (e, 0)),
            pl.BlockSpec((None, NKW, D), lambda e, n, c: (e, n, 0)),
            pl.BlockSpec((None, NKW, D), lambda e, n, c: (e, NCH + n, 0)),
            pl.BlockSpec((None, D, NKW), lambda e, n, c: (e, 0, n)),
            pl.BlockSpec((None, 1, NKW), lambda e, n, c: (e * 2 * NCH + n, 0, 0)),
            pl.BlockSpec((None, 1, NKW),
                         lambda e, n, c: (e * 2 * NCH + NCH + n, 0, 0)),
            pl.BlockSpec((None, 1, D), lambda e, n, c: (e, 0, 0)),
        ],
        out_specs=pl.BlockSpec((CAP, D), lambda e, n, c: (e, 0)),
        scratch_shapes=[pltpu.VMEM((NKW, D), _bf16),
                        pltpu.VMEM((NKW, D), _bf16),
                        pltpu.VMEM((D, NKW), _bf16)],
    )
    ff = pl.pallas_call(
        _ffn_body,
        grid_spec=grid_spec,
        out_shape=jax.ShapeDtypeStruct((E * CAP, D), _f32),
        compiler_params=pltpu.CompilerParams(
            dimension_semantics=("parallel", "arbitrary")),
    )(counts, xs, W_in, W_in, W_out,
      b_in.reshape(E * 2 * NCH, 1, NKW), b_in.reshape(E * 2 * NCH, 1, NKW),
      b_out.reshape(E, 1, D))

    @pl.kernel(out_type=jax.ShapeDtypeStruct((4 * T, HD), _f32), mesh=vmesh)
    def _gather_k(ff_hbm, i_hbm, g_hbm):
        def body(i_vmem, g_vmem):
            pltpu.sync_copy(ff_hbm.at[i_vmem.at[0]], g_vmem)

        pltpu.emit_pipeline(
            body,
            grid=(4 * T // SCW,),
            in_specs=[pl.BlockSpec((1, SCW), lambda i: (i, 0))],
            out_specs=[pl.BlockSpec((SCW, HD), lambda i: (i, 0))],
            core_axis_name=("c", "s"),
            dimension_semantics=(pltpu.PARALLEL,),
        )(i_hbm, g_hbm)

    g = _gather_k(ff.reshape(2 * E * CAP, HD), idx2).reshape(2 * T, D)

    out = pl.pallas_call(
        _comb_body,
        out_shape=jax.ShapeDtypeStruct((T, D), _f32),
    )(w0c, w1c, g[:T], g[T:])
    return out


# speedup vs baseline: 2.0765x; 1.0018x over previous
import jax
import jax.numpy as jnp
from jax.experimental import pallas as pl
from jax.experimental.pallas import tpu as pltpu
from jax.experimental.pallas import tpu_sc as plsc

T = 1024
D = 768
I = 3072
E = 8
CAP = T
TM = 256
NKW = 1536
NCH = I // NKW
SCW = 128

_f32 = jnp.float32
_bf16 = jnp.bfloat16


def _router_body(y_ref, dest_ref, w0_ref, w1_ref, cnt_ref):
    y = y_ref[...]
    iota = jax.lax.broadcasted_iota(jnp.int32, (T, E), 1)
    v0 = jnp.max(y, axis=1, keepdims=True)
    i0 = jnp.min(jnp.where(y == v0, iota, E), axis=1, keepdims=True)
    y2 = jnp.where(iota == i0, -jnp.inf, y)
    v1 = jnp.max(y2, axis=1, keepdims=True)
    i1 = jnp.min(jnp.where(y2 == v1, iota, E), axis=1, keepdims=True)
    e1 = jnp.exp(v1 - v0)
    den = 1.0 + e1
    a0 = (iota == i0).astype(_f32)
    a1 = (iota == i1).astype(_f32)
    rows = jax.lax.broadcasted_iota(jnp.int32, (T, T), 0)
    cols = jax.lax.broadcasted_iota(jnp.int32, (T, T), 1)
    tri = (cols < rows).astype(_f32)
    c0 = jax.lax.dot_general(tri, a0, (((1,), (0,)), ((), ())),
                             preferred_element_type=_f32)
    c1 = jax.lax.dot_general(tri, a1, (((1,), (0,)), ((), ())),
                             preferred_element_type=_f32)
    s0 = jnp.sum(a0, axis=0, keepdims=True)
    s1 = jnp.sum(a1, axis=0, keepdims=True)
    r0 = jnp.sum(c0 * a0, axis=1)
    r1 = jnp.sum((c1 + s0) * a1, axis=1)
    dest_ref[0, :] = i0[:, 0] * CAP + r0.astype(jnp.int32)
    dest_ref[1, :] = i1[:, 0] * CAP + r1.astype(jnp.int32)
    w0_ref[...] = 1.0 / den
    w1_ref[...] = e1 / den
    cnt_ref[0, :] = (s0 + s1)[0, :].astype(jnp.int32)


def _ffn_body(cnt_ref, xs_ref, w1a_ref, w1b_ref, wo_ref, bina_ref, binb_ref,
              bout_ref, ff_ref):
    n = pl.program_id(1)
    c = cnt_ref[pl.program_id(0)]
    nt = (c + TM - 1) // TM

    def tile(mi, carry):
        sl = pl.ds(mi * TM, TM)
        xv = xs_ref[sl, :]
        h1 = jax.lax.dot_general(xv, w1a_ref[...], (((1,), (1,)), ((), ())),
                                 preferred_element_type=_f32) + bina_ref[...]
        h2 = jax.lax.dot_general(xv, w1b_ref[...], (((1,), (1,)), ((), ())),
                                 preferred_element_type=_f32) + binb_ref[...]
        act = h1 * (1.0 / (1.0 + jnp.exp(-h2)))
        o = jax.lax.dot_general(act, wo_ref[...], (((1,), (1,)), ((), ())),
                                preferred_element_type=_f32)

        @pl.when(n == 0)
        def _():
            ff_ref[sl, :] = o + bout_ref[...]

        @pl.when(n > 0)
        def _():
            ff_ref[sl, :] += o

        return carry

    jax.lax.fori_loop(0, nt, tile, 0)


def _comb_body(w0_ref, w1_ref, g0_ref, g1_ref, o_ref):
    o_ref[...] = w0_ref[...] * g0_ref[...] + w1_ref[...] * g1_ref[...]


def kernel(x, W_r, b_r, W_in, b_in, W_out, b_out):
    logits = x @ W_r.T + b_r
    u = jax.random.uniform(jax.random.key(42), logits.shape, dtype=logits.dtype)
    noise = -jnp.log(-jnp.log(u + 1e-20) + 1e-20)
    y = (logits + noise) / 1.0

    dest, w0c, w1c, cnt = pl.pallas_call(
        _router_body,
        out_shape=(
            jax.ShapeDtypeStruct((2, T), jnp.int32),
            jax.ShapeDtypeStruct((T, 1), _f32),
            jax.ShapeDtypeStruct((T, 1), _f32),
            jax.ShapeDtypeStruct((1, E), jnp.int32),
        ),
    )(y)
    counts = cnt.reshape(E)
    idx2 = (2 * dest.reshape(2 * T, 1)
            + jax.lax.broadcasted_iota(jnp.int32, (2 * T, 2), 1))
    idx2 = idx2.reshape(4 * T // SCW, SCW)

    vmesh = plsc.VectorSubcoreMesh(core_axis_name="c", subcore_axis_name="s",
                                   num_cores=2, num_subcores=16)
    HD = D // 2

    @pl.kernel(out_type=jax.ShapeDtypeStruct((2 * E * CAP, HD), _f32),
               mesh=vmesh)
    def _scatter_k(x_hbm, i_hbm, xs_hbm):
        def body(x_vmem, i_vmem):
            pltpu.sync_copy(x_vmem, xs_hbm.at[i_vmem.at[0]])

        pltpu.emit_pipeline(
            body,
            grid=(4 * T // SCW,),
            in_specs=[
                pl.BlockSpec((SCW, HD), lambda i: (i % (2 * T // SCW), 0)),
                pl.BlockSpec((1, SCW), lambda i: (i, 0)),
            ],
            out_specs=[],
            core_axis_name=("c", "s"),
            dimension_semantics=(pltpu.PARALLEL,),
        )(x_hbm, i_hbm)

    xs = _scatter_k(x.reshape(2 * T, HD), idx2).reshape(E * CAP, D)

    grid_spec = pltpu.PrefetchScalarGridSpec(
        num_scalar_prefetch=1,
        grid=(E, NCH),
        in_specs=[
            pl.BlockSpec((CAP, D), lambda e, n, c: (e, 0)),
            pl.BlockSpec((None, NKW, D), lambda e, n, c: (e, n, 0)),
            pl.BlockSpec((None, NKW, D), lambda e, n, c: (e, NCH + n, 0)),
            pl.BlockSpec((None, D, NKW), lambda e, n, c: (e, 0, n)),
            pl.BlockSpec((None, 1, NKW), lambda e, n, c: (e * 2 * NCH + n, 0, 0)),
            pl.BlockSpec((None, 1, NKW),
                         lambda e, n, c: (e * 2 * NCH + NCH + n, 0, 0)),
            pl.BlockSpec((None, 1, D), lambda e, n, c: (e, 0, 0)),
        ],
        out_specs=pl.BlockSpec((CAP, D), lambda e, n, c: (e, 0)),
    )
    ff = pl.pallas_call(
        _ffn_body,
        grid_spec=grid_spec,
        out_shape=jax.ShapeDtypeStruct((E * CAP, D), _f32),
        compiler_params=pltpu.CompilerParams(
            dimension_semantics=("parallel", "arbitrary")),
    )(counts, xs, W_in, W_in, W_out,
      b_in.reshape(E * 2 * NCH, 1, NKW), b_in.reshape(E * 2 * NCH, 1, NKW),
      b_out.reshape(E, 1, D))

    @pl.kernel(out_type=jax.ShapeDtypeStruct((4 * T, HD), _f32), mesh=vmesh)
    def _gather_k(ff_hbm, i_hbm, g_hbm):
        def body(i_vmem, g_vmem):
            pltpu.sync_copy(ff_hbm.at[i_vmem.at[0]], g_vmem)

        pltpu.emit_pipeline(
            body,
            grid=(4 * T // SCW,),
            in_specs=[pl.BlockSpec((1, SCW), lambda i: (i, 0))],
            out_specs=[pl.BlockSpec((SCW, HD), lambda i: (i, 0))],
            core_axis_name=("c", "s"),
            dimension_semantics=(pltpu.PARALLEL,),
        )(i_hbm, g_hbm)

    g = _gather_k(ff.reshape(2 * E * CAP, HD), idx2).reshape(2 * T, D)

    out = pl.pallas_call(
        _comb_body,
        out_shape=jax.ShapeDtypeStruct((T, D), _f32),
    )(w0c, w1c, g[:T], g[T:])
    return out
